# baseline (speedup 1.0000x reference)
import functools

import jax
import jax.numpy as jnp
from jax import lax
from jax.experimental import pallas as pl
from jax.experimental.pallas import tpu as pltpu
from jax.experimental.pallas import tpu_sc as plsc

N = 512
E = 8192
P = 4096
M = 20
EBLK = 2048
QBLK = 512
HQB = QBLK // 2
NEB = E // EBLK
NQB = (2 * P) // QBLK
NB = N // 8
TD = 128

_NC = 2
_NS = 16
_NW = _NC * _NS
_BPW = (2 * P) // _NW


def _k1(row_ref, col_ref, feat_ref, w0_ref, b0_ref, w1_ref, b1_ref,
        wm1_ref, bm1_ref, wm2_ref, bm2_ref,
        adjf_ref, adjt_ref, ti_ref, tj_ref, v1_ref, u2_ref, w_ref):
    e = pl.program_id(0)
    rows = row_ref[0, 0, :]
    cols = col_ref[0, 0, :]
    ids = jax.lax.broadcasted_iota(jnp.int32, (EBLK, N), 1)
    ohr = (rows[:, None] == ids).astype(jnp.bfloat16)
    ohc = (cols[:, None] == ids).astype(jnp.bfloat16)
    dn = (((0,), (0,)), ((), ()))
    pa = jax.lax.dot_general(ohr, ohc, dn,
                             preferred_element_type=jnp.float32
                             ).astype(jnp.bfloat16)
    pt = jax.lax.dot_general(ohc, ohr, dn,
                             preferred_element_type=jnp.float32
                             ).astype(jnp.bfloat16)

    @pl.when(e == 0)
    def _():
        adjf_ref[...] = pa
        adjt_ref[...] = pt

    @pl.when(e > 0)
    def _():
        adjf_ref[...] += pa
        adjt_ref[...] += pt

    @pl.when(e == NEB - 1)
    def _():
        f32 = jnp.float32
        adjtf = adjt_ref[...].astype(f32)
        deg = jnp.sum(adjtf, axis=1) + 1.0
        dinv = 1.0 / jnp.sqrt(deg)
        ir = jax.lax.broadcasted_iota(jnp.int32, (N, N), 0)
        ic = jax.lax.broadcasted_iota(jnp.int32, (N, N), 1)
        eye = (ir == ic).astype(f32)
        prop = (adjtf + eye) * (dinv[:, None] * dinv[None, :])
        h = jnp.dot(prop, jnp.dot(feat_ref[...], w0_ref[...],
                                  preferred_element_type=f32),
                    preferred_element_type=f32) + b0_ref[...]
        h = jnp.dot(prop, jnp.dot(h, w1_ref[...],
                                  preferred_element_type=f32),
                    preferred_element_type=f32) + b1_ref[...]
        wm1 = wm1_ref[...]
        wm2 = wm2_ref[...]
        u1 = jnp.dot(h, wm1[:M, :], preferred_element_type=f32) + bm1_ref[...]
        v1 = jnp.dot(h, wm1[M:, :], preferred_element_type=f32)
        u2 = jnp.dot(h, wm2[:M, :], preferred_element_type=f32)
        v2 = jnp.dot(h, wm2[M:, :], preferred_element_type=f32) + bm2_ref[...]
        v1_ref[...] = v1
        u2_ref[...] = u2
        w_ref[...] = v1 * u2
        icc = jax.lax.broadcasted_iota(jnp.int32, (N, NB), 0)
        iw = jax.lax.broadcasted_iota(jnp.int32, (N, NB), 1)
        packm = jnp.where((icc >> 3) == iw, (1 << (icc & 7)), 0).astype(f32)
        smb = jnp.dot(adjf_ref[...].astype(f32), packm,
                      preferred_element_type=f32)
        pmb = jnp.dot(adjtf, packm, preferred_element_type=f32)
        zpad = jnp.zeros((N, 32 - M), f32)
        ti_ref[...] = jnp.concatenate([u1, zpad, h, zpad, smb], axis=1)
        tj_ref[...] = jnp.concatenate([v2, zpad, h, zpad, pmb], axis=1)


def _sc_gather(ti_hbm, tj_hbm, iq_hbm, jq_hbm, gi_hbm, gj_hbm,
               idx_v, rows_v, sem):
    wid = lax.axis_index("s") * _NC + lax.axis_index("c")
    base = wid * _BPW
    pltpu.sync_copy(iq_hbm.at[pl.ds(base, _BPW)], idx_v)
    pltpu.async_copy(ti_hbm.at[idx_v], rows_v, sem).wait()
    pltpu.sync_copy(rows_v, gi_hbm.at[pl.ds(base, _BPW)])
    pltpu.sync_copy(jq_hbm.at[pl.ds(base, _BPW)], idx_v)
    pltpu.async_copy(tj_hbm.at[idx_v], rows_v, sem).wait()
    pltpu.sync_copy(rows_v, gj_hbm.at[pl.ds(base, _BPW)])


def _k2(jq_ref, gi_ref, gj_ref, v1_ref, u2_ref, w_ref,
        wm3_ref, bm3_ref, wdir_ref, bdir_ref, out_ref):
    f32 = jnp.float32
    jqv = jq_ref[0, 0, :]
    gi = gi_ref[...]
    gj = gj_ref[...]
    u1q = gi[:, 0:M]
    hr = gi[:, 32:32 + M]
    v2q = gj[:, 0:M]
    hc = gj[:, 32:32 + M]
    ibw = jax.lax.broadcasted_iota(jnp.int32, (NB, N), 0)
    icc = jax.lax.broadcasted_iota(jnp.int32, (NB, N), 1)
    dmat = (ibw == (icc >> 3)).astype(f32)
    ilane = jax.lax.broadcasted_iota(jnp.int32, (1, N), 1)
    shinv = (1.0 / (1 << (ilane & 7)).astype(f32))
    bi = jnp.floor(jnp.dot(gi[:, 64:], dmat, preferred_element_type=f32)
                   * shinv)
    aqr = bi - 2.0 * jnp.floor(bi * 0.5)
    bj = jnp.floor(jnp.dot(gj[:, 64:], dmat, preferred_element_type=f32)
                   * shinv)
    aqc = bj - 2.0 * jnp.floor(bj * 0.5)
    z = aqr * aqc
    cnt = jnp.sum(z, axis=1)
    s1 = jnp.dot(z, v1_ref[...], preferred_element_type=f32)
    s2 = jnp.dot(z, u2_ref[...], preferred_element_type=f32)
    sx = jnp.dot(z, w_ref[...], preferred_element_type=f32)
    ids = jax.lax.broadcasted_iota(jnp.int32, (QBLK, N), 1)
    ohjf = (jqv[:, None] == ids).astype(f32)
    adjflag = jnp.sum(aqr * ohjf, axis=1)
    pd = (u1q * v2q) * cnt[:, None] + u1q * s2 + v2q * s1 + sx
    union = ((cnt + adjflag) > 0.0).astype(f32)
    wm3 = wm3_ref[...]
    aval = ((jnp.dot(pd, wm3[:M, :], preferred_element_type=f32)
             + adjflag[:, None] * wm3[M, :][None, :] + bm3_ref[...])
            * union[:, None])
    xf = aval[0:HQB, :] * aval[HQB:QBLK, :]
    xx = hr[0:HQB, :] * hc[0:HQB, :]
    wdir = wdir_ref[...]
    out_ref[...] = (jnp.dot(xf, wdir[:M, :], preferred_element_type=f32)
                    + jnp.dot(xx, wdir[M:, :], preferred_element_type=f32)
                    + bdir_ref[...])


def kernel(x, ei, pos, feat, W0, b0, W1, b1, Wm1, bm1, Wm2, bm2,
           Wm3, bm3, Wdir, bdir):
    f32 = jnp.float32
    row = ei[0].reshape(NEB, 1, EBLK)
    col = ei[1].reshape(NEB, 1, EBLK)
    iq = jnp.concatenate([pos[:, 0].reshape(NQB, 1, HQB),
                          pos[:, 1].reshape(NQB, 1, HQB)], axis=2)
    jq = jnp.concatenate([pos[:, 1].reshape(NQB, 1, HQB),
                          pos[:, 0].reshape(NQB, 1, HQB)], axis=2)

    full = lambda shp: pl.BlockSpec(shp, lambda *_: tuple(0 for _ in shp))
    ebk = pl.BlockSpec((1, 1, EBLK), lambda e: (e, 0, 0))
    qbk = pl.BlockSpec((1, 1, QBLK), lambda q: (q, 0, 0))

    adjf, adjt, ti, tj, v1, u2, w = pl.pallas_call(
        _k1,
        grid=(NEB,),
        in_specs=[ebk, ebk, full((N, 128)), full((128, M)), full((1, M)),
                  full((M, M)), full((1, M)), full((2 * M, M)), full((1, M)),
                  full((2 * M, M)), full((1, M))],
        out_specs=[full((N, N)), full((N, N)), full((N, TD)), full((N, TD))]
        + [full((N, M))] * 3,
        out_shape=[jax.ShapeDtypeStruct((N, N), jnp.bfloat16)] * 2
        + [jax.ShapeDtypeStruct((N, TD), f32)] * 2
        + [jax.ShapeDtypeStruct((N, M), f32)] * 3,
    )(row, col, feat, W0, b0.reshape(1, M), W1, b1.reshape(1, M),
      Wm1, bm1.reshape(1, M), Wm2, bm2.reshape(1, M))

    sc = functools.partial(
        pl.kernel,
        out_type=[jax.ShapeDtypeStruct((2 * P, TD), f32)] * 2,
        mesh=plsc.VectorSubcoreMesh(core_axis_name="c", subcore_axis_name="s"),
        scratch_types=[pltpu.VMEM((_BPW,), jnp.int32),
                       pltpu.VMEM((_BPW, TD), f32),
                       pltpu.SemaphoreType.DMA],
    )(_sc_gather)
    gi, gj = sc(ti, tj, iq.reshape(-1), jq.reshape(-1))

    out = pl.pallas_call(
        _k2,
        grid=(NQB,),
        in_specs=[qbk,
                  pl.BlockSpec((QBLK, TD), lambda q: (q, 0)),
                  pl.BlockSpec((QBLK, TD), lambda q: (q, 0))]
        + [full((N, M))] * 3
        + [full((M + 1, M)), full((1, M)), full((2 * M, 1)), full((1, 1))],
        out_specs=pl.BlockSpec((HQB, 1), lambda q: (q, 0)),
        out_shape=jax.ShapeDtypeStruct((P, 1), f32),
    )(jq, gi, gj, v1, u2, w, Wm3, bm3.reshape(1, M),
      Wdir, bdir.reshape(1, 1))
    return out


# speedup vs baseline: 1.0252x; 1.0252x over previous
import functools

import jax
import jax.numpy as jnp
from jax import lax
from jax.experimental import pallas as pl
from jax.experimental.pallas import tpu as pltpu
from jax.experimental.pallas import tpu_sc as plsc

N = 512
E = 8192
P = 4096
M = 20
EBLK = 2048
QBLK = 512
HQB = QBLK // 2
NEB = E // EBLK
NQB = (2 * P) // QBLK
NB = N // 8
TD = 128

_NC = 2
_NS = 16
_NW = _NC * _NS
_BPW = (2 * P) // _NW


def _k1(row_ref, col_ref, feat_ref, w0_ref, b0_ref, w1_ref, b1_ref,
        wm1_ref, bm1_ref, wm2_ref, bm2_ref,
        adjf_ref, adjt_ref, ti_ref, tj_ref, v1_ref, u2_ref, w_ref):
    e = pl.program_id(0)
    rows = row_ref[0, 0, :]
    cols = col_ref[0, 0, :]
    ids = jax.lax.broadcasted_iota(jnp.int32, (EBLK, N), 1)
    ohr = (rows[:, None] == ids).astype(jnp.bfloat16)
    ohc = (cols[:, None] == ids).astype(jnp.bfloat16)
    dn = (((0,), (0,)), ((), ()))
    pa = jax.lax.dot_general(ohr, ohc, dn,
                             preferred_element_type=jnp.float32
                             ).astype(jnp.bfloat16)
    pt = jax.lax.dot_general(ohc, ohr, dn,
                             preferred_element_type=jnp.float32
                             ).astype(jnp.bfloat16)

    @pl.when(e == 0)
    def _():
        adjf_ref[...] = pa
        adjt_ref[...] = pt

    @pl.when(e > 0)
    def _():
        adjf_ref[...] += pa
        adjt_ref[...] += pt

    @pl.when(e == NEB - 1)
    def _():
        f32 = jnp.float32
        adjtf = adjt_ref[...].astype(f32)
        deg = jnp.sum(adjtf, axis=1) + 1.0
        dinv = 1.0 / jnp.sqrt(deg)
        ir = jax.lax.broadcasted_iota(jnp.int32, (N, N), 0)
        ic = jax.lax.broadcasted_iota(jnp.int32, (N, N), 1)
        eye = (ir == ic).astype(f32)
        prop = (adjtf + eye) * (dinv[:, None] * dinv[None, :])
        h = jnp.dot(prop, jnp.dot(feat_ref[...], w0_ref[...],
                                  preferred_element_type=f32),
                    preferred_element_type=f32) + b0_ref[...]
        h = jnp.dot(prop, jnp.dot(h, w1_ref[...],
                                  preferred_element_type=f32),
                    preferred_element_type=f32) + b1_ref[...]
        wm1 = wm1_ref[...]
        wm2 = wm2_ref[...]
        u1 = jnp.dot(h, wm1[:M, :], preferred_element_type=f32) + bm1_ref[...]
        v1 = jnp.dot(h, wm1[M:, :], preferred_element_type=f32)
        u2 = jnp.dot(h, wm2[:M, :], preferred_element_type=f32)
        v2 = jnp.dot(h, wm2[M:, :], preferred_element_type=f32) + bm2_ref[...]
        v1_ref[...] = v1
        u2_ref[...] = u2
        w_ref[...] = v1 * u2
        icc = jax.lax.broadcasted_iota(jnp.int32, (N, NB), 0)
        iw = jax.lax.broadcasted_iota(jnp.int32, (N, NB), 1)
        packm = jnp.where((icc >> 3) == iw, (1 << (icc & 7)), 0).astype(f32)
        smb = jnp.dot(adjf_ref[...].astype(f32), packm,
                      preferred_element_type=f32)
        pmb = jnp.dot(adjtf, packm, preferred_element_type=f32)
        zpad = jnp.zeros((N, 32 - M), f32)
        ti_ref[...] = jnp.concatenate([u1, zpad, h, zpad, smb], axis=1)
        tj_ref[...] = jnp.concatenate([v2, zpad, h, zpad, pmb], axis=1)


def _sc_gather(ti_hbm, tj_hbm, iq_hbm, jq_hbm, gi_hbm, gj_hbm,
               idxi_v, idxj_v, rowsi_v, rowsj_v, semi, semj):
    wid = lax.axis_index("s") * _NC + lax.axis_index("c")
    base = wid * _BPW
    pltpu.sync_copy(iq_hbm.at[pl.ds(base, _BPW)], idxi_v)
    pltpu.sync_copy(jq_hbm.at[pl.ds(base, _BPW)], idxj_v)
    ci = pltpu.async_copy(ti_hbm.at[idxi_v], rowsi_v, semi)
    cj = pltpu.async_copy(tj_hbm.at[idxj_v], rowsj_v, semj)
    ci.wait()
    pltpu.sync_copy(rowsi_v, gi_hbm.at[pl.ds(base, _BPW)])
    cj.wait()
    pltpu.sync_copy(rowsj_v, gj_hbm.at[pl.ds(base, _BPW)])


def _k2(jq_ref, gi_ref, gj_ref, v1_ref, u2_ref, w_ref,
        wm3_ref, bm3_ref, wdir_ref, bdir_ref, out_ref):
    f32 = jnp.float32
    jqv = jq_ref[0, 0, :]
    gi = gi_ref[...]
    gj = gj_ref[...]
    u1q = gi[:, 0:M]
    hr = gi[:, 32:32 + M]
    v2q = gj[:, 0:M]
    hc = gj[:, 32:32 + M]
    ibw = jax.lax.broadcasted_iota(jnp.int32, (NB, N), 0)
    icc = jax.lax.broadcasted_iota(jnp.int32, (NB, N), 1)
    dmat = (ibw == (icc >> 3)).astype(f32)
    ilane = jax.lax.broadcasted_iota(jnp.int32, (1, N), 1)
    shinv = (1.0 / (1 << (ilane & 7)).astype(f32))
    bi = jnp.floor(jnp.dot(gi[:, 64:], dmat, preferred_element_type=f32)
                   * shinv)
    aqr = bi - 2.0 * jnp.floor(bi * 0.5)
    bj = jnp.floor(jnp.dot(gj[:, 64:], dmat, preferred_element_type=f32)
                   * shinv)
    aqc = bj - 2.0 * jnp.floor(bj * 0.5)
    z = aqr * aqc
    cnt = jnp.sum(z, axis=1)
    s1 = jnp.dot(z, v1_ref[...], preferred_element_type=f32)
    s2 = jnp.dot(z, u2_ref[...], preferred_element_type=f32)
    sx = jnp.dot(z, w_ref[...], preferred_element_type=f32)
    ids = jax.lax.broadcasted_iota(jnp.int32, (QBLK, N), 1)
    ohjf = (jqv[:, None] == ids).astype(f32)
    adjflag = jnp.sum(aqr * ohjf, axis=1)
    pd = (u1q * v2q) * cnt[:, None] + u1q * s2 + v2q * s1 + sx
    union = ((cnt + adjflag) > 0.0).astype(f32)
    wm3 = wm3_ref[...]
    aval = ((jnp.dot(pd, wm3[:M, :], preferred_element_type=f32)
             + adjflag[:, None] * wm3[M, :][None, :] + bm3_ref[...])
            * union[:, None])
    xf = aval[0:HQB, :] * aval[HQB:QBLK, :]
    xx = hr[0:HQB, :] * hc[0:HQB, :]
    wdir = wdir_ref[...]
    out_ref[...] = (jnp.dot(xf, wdir[:M, :], preferred_element_type=f32)
                    + jnp.dot(xx, wdir[M:, :], preferred_element_type=f32)
                    + bdir_ref[...])


def kernel(x, ei, pos, feat, W0, b0, W1, b1, Wm1, bm1, Wm2, bm2,
           Wm3, bm3, Wdir, bdir):
    f32 = jnp.float32
    row = ei[0].reshape(NEB, 1, EBLK)
    col = ei[1].reshape(NEB, 1, EBLK)
    iq = jnp.concatenate([pos[:, 0].reshape(NQB, 1, HQB),
                          pos[:, 1].reshape(NQB, 1, HQB)], axis=2)
    jq = jnp.concatenate([pos[:, 1].reshape(NQB, 1, HQB),
                          pos[:, 0].reshape(NQB, 1, HQB)], axis=2)

    full = lambda shp: pl.BlockSpec(shp, lambda *_: tuple(0 for _ in shp))
    ebk = pl.BlockSpec((1, 1, EBLK), lambda e: (e, 0, 0))
    qbk = pl.BlockSpec((1, 1, QBLK), lambda q: (q, 0, 0))

    adjf, adjt, ti, tj, v1, u2, w = pl.pallas_call(
        _k1,
        grid=(NEB,),
        in_specs=[ebk, ebk, full((N, 128)), full((128, M)), full((1, M)),
                  full((M, M)), full((1, M)), full((2 * M, M)), full((1, M)),
                  full((2 * M, M)), full((1, M))],
        out_specs=[full((N, N)), full((N, N)), full((N, TD)), full((N, TD))]
        + [full((N, M))] * 3,
        out_shape=[jax.ShapeDtypeStruct((N, N), jnp.bfloat16)] * 2
        + [jax.ShapeDtypeStruct((N, TD), f32)] * 2
        + [jax.ShapeDtypeStruct((N, M), f32)] * 3,
    )(row, col, feat, W0, b0.reshape(1, M), W1, b1.reshape(1, M),
      Wm1, bm1.reshape(1, M), Wm2, bm2.reshape(1, M))

    sc = functools.partial(
        pl.kernel,
        out_type=[jax.ShapeDtypeStruct((2 * P, TD), f32)] * 2,
        mesh=plsc.VectorSubcoreMesh(core_axis_name="c", subcore_axis_name="s"),
        scratch_types=[pltpu.VMEM((_BPW,), jnp.int32),
                       pltpu.VMEM((_BPW,), jnp.int32),
                       pltpu.VMEM((_BPW, TD), f32),
                       pltpu.VMEM((_BPW, TD), f32),
                       pltpu.SemaphoreType.DMA,
                       pltpu.SemaphoreType.DMA],
    )(_sc_gather)
    gi, gj = sc(ti, tj, iq.reshape(-1), jq.reshape(-1))

    out = pl.pallas_call(
        _k2,
        grid=(NQB,),
        in_specs=[qbk,
                  pl.BlockSpec((QBLK, TD), lambda q: (q, 0)),
                  pl.BlockSpec((QBLK, TD), lambda q: (q, 0))]
        + [full((N, M))] * 3
        + [full((M + 1, M)), full((1, M)), full((2 * M, 1)), full((1, 1))],
        out_specs=pl.BlockSpec((HQB, 1), lambda q: (q, 0)),
        out_shape=jax.ShapeDtypeStruct((P, 1), f32),
    )(jq, gi, gj, v1, u2, w, Wm3, bm3.reshape(1, M),
      Wdir, bdir.reshape(1, 1))
    return out


# speedup vs baseline: 1.0887x; 1.0620x over previous
import functools

import jax
import jax.numpy as jnp
from jax import lax
from jax.experimental import pallas as pl
from jax.experimental.pallas import tpu as pltpu
from jax.experimental.pallas import tpu_sc as plsc

N = 512
E = 8192
P = 4096
M = 20
EBLK = 4096
QBLK = 1024
HQB = QBLK // 2
NEB = E // EBLK
NQB = (2 * P) // QBLK
NB = N // 8
TD = 128

_NC = 2
_NS = 16
_NW = _NC * _NS
_BPW = (2 * P) // _NW


def _k1(row_ref, col_ref, feat_ref, w0_ref, b0_ref, w1_ref, b1_ref,
        wm1_ref, bm1_ref, wm2_ref, bm2_ref,
        adjf_ref, adjt_ref, ti_ref, tj_ref, v1_ref, u2_ref, w_ref):
    e = pl.program_id(0)
    rows = row_ref[0, 0, :]
    cols = col_ref[0, 0, :]
    ids = jax.lax.broadcasted_iota(jnp.int32, (EBLK, N), 1)
    ohr = (rows[:, None] == ids).astype(jnp.bfloat16)
    ohc = (cols[:, None] == ids).astype(jnp.bfloat16)
    dn = (((0,), (0,)), ((), ()))
    pa = jax.lax.dot_general(ohr, ohc, dn,
                             preferred_element_type=jnp.float32
                             ).astype(jnp.bfloat16)
    pt = jax.lax.dot_general(ohc, ohr, dn,
                             preferred_element_type=jnp.float32
                             ).astype(jnp.bfloat16)

    @pl.when(e == 0)
    def _():
        adjf_ref[...] = pa
        adjt_ref[...] = pt

    @pl.when(e > 0)
    def _():
        adjf_ref[...] += pa
        adjt_ref[...] += pt

    @pl.when(e == NEB - 1)
    def _():
        f32 = jnp.float32
        adjtf = adjt_ref[...].astype(f32)
        deg = jnp.sum(adjtf, axis=1) + 1.0
        dinv = 1.0 / jnp.sqrt(deg)
        ir = jax.lax.broadcasted_iota(jnp.int32, (N, N), 0)
        ic = jax.lax.broadcasted_iota(jnp.int32, (N, N), 1)
        eye = (ir == ic).astype(f32)
        prop = (adjtf + eye) * (dinv[:, None] * dinv[None, :])
        h = jnp.dot(prop, jnp.dot(feat_ref[...], w0_ref[...],
                                  preferred_element_type=f32),
                    preferred_element_type=f32) + b0_ref[...]
        h = jnp.dot(prop, jnp.dot(h, w1_ref[...],
                                  preferred_element_type=f32),
                    preferred_element_type=f32) + b1_ref[...]
        wm1 = wm1_ref[...]
        wm2 = wm2_ref[...]
        u1 = jnp.dot(h, wm1[:M, :], preferred_element_type=f32) + bm1_ref[...]
        v1 = jnp.dot(h, wm1[M:, :], preferred_element_type=f32)
        u2 = jnp.dot(h, wm2[:M, :], preferred_element_type=f32)
        v2 = jnp.dot(h, wm2[M:, :], preferred_element_type=f32) + bm2_ref[...]
        v1_ref[...] = v1
        u2_ref[...] = u2
        w_ref[...] = v1 * u2
        icc = jax.lax.broadcasted_iota(jnp.int32, (N, NB), 0)
        iw = jax.lax.broadcasted_iota(jnp.int32, (N, NB), 1)
        packm = jnp.where((icc >> 3) == iw, (1 << (icc & 7)), 0).astype(f32)
        smb = jnp.dot(adjf_ref[...].astype(f32), packm,
                      preferred_element_type=f32)
        pmb = jnp.dot(adjtf, packm, preferred_element_type=f32)
        zpad = jnp.zeros((N, 32 - M), f32)
        ti_ref[...] = jnp.concatenate([u1, zpad, h, zpad, smb], axis=1)
        tj_ref[...] = jnp.concatenate([v2, zpad, h, zpad, pmb], axis=1)


def _sc_gather(ti_hbm, tj_hbm, iq_hbm, jq_hbm, gi_hbm, gj_hbm,
               idxi_v, idxj_v, rowsi_v, rowsj_v, semi, semj):
    wid = lax.axis_index("s") * _NC + lax.axis_index("c")
    base = wid * _BPW
    pltpu.sync_copy(iq_hbm.at[pl.ds(base, _BPW)], idxi_v)
    pltpu.sync_copy(jq_hbm.at[pl.ds(base, _BPW)], idxj_v)
    ci = pltpu.async_copy(ti_hbm.at[idxi_v], rowsi_v, semi)
    cj = pltpu.async_copy(tj_hbm.at[idxj_v], rowsj_v, semj)
    ci.wait()
    pltpu.sync_copy(rowsi_v, gi_hbm.at[pl.ds(base, _BPW)])
    cj.wait()
    pltpu.sync_copy(rowsj_v, gj_hbm.at[pl.ds(base, _BPW)])


def _k2(jq_ref, gi_ref, gj_ref, v1_ref, u2_ref, w_ref,
        wm3_ref, bm3_ref, wdir_ref, bdir_ref, out_ref):
    f32 = jnp.float32
    jqv = jq_ref[0, 0, :]
    gi = gi_ref[...]
    gj = gj_ref[...]
    u1q = gi[:, 0:M]
    hr = gi[:, 32:32 + M]
    v2q = gj[:, 0:M]
    hc = gj[:, 32:32 + M]
    ibw = jax.lax.broadcasted_iota(jnp.int32, (NB, N), 0)
    icc = jax.lax.broadcasted_iota(jnp.int32, (NB, N), 1)
    dmat = (ibw == (icc >> 3)).astype(f32)
    ilane = jax.lax.broadcasted_iota(jnp.int32, (1, N), 1)
    shinv = (1.0 / (1 << (ilane & 7)).astype(f32))
    bi = jnp.floor(jnp.dot(gi[:, 64:], dmat, preferred_element_type=f32)
                   * shinv)
    aqr = bi - 2.0 * jnp.floor(bi * 0.5)
    bj = jnp.floor(jnp.dot(gj[:, 64:], dmat, preferred_element_type=f32)
                   * shinv)
    aqc = bj - 2.0 * jnp.floor(bj * 0.5)
    z = aqr * aqc
    cnt = jnp.sum(z, axis=1)
    s1 = jnp.dot(z, v1_ref[...], preferred_element_type=f32)
    s2 = jnp.dot(z, u2_ref[...], preferred_element_type=f32)
    sx = jnp.dot(z, w_ref[...], preferred_element_type=f32)
    ids = jax.lax.broadcasted_iota(jnp.int32, (QBLK, N), 1)
    ohjf = (jqv[:, None] == ids).astype(f32)
    adjflag = jnp.sum(aqr * ohjf, axis=1)
    pd = (u1q * v2q) * cnt[:, None] + u1q * s2 + v2q * s1 + sx
    union = ((cnt + adjflag) > 0.0).astype(f32)
    wm3 = wm3_ref[...]
    aval = ((jnp.dot(pd, wm3[:M, :], preferred_element_type=f32)
             + adjflag[:, None] * wm3[M, :][None, :] + bm3_ref[...])
            * union[:, None])
    xf = aval[0:HQB, :] * aval[HQB:QBLK, :]
    xx = hr[0:HQB, :] * hc[0:HQB, :]
    wdir = wdir_ref[...]
    out_ref[...] = (jnp.dot(xf, wdir[:M, :], preferred_element_type=f32)
                    + jnp.dot(xx, wdir[M:, :], preferred_element_type=f32)
                    + bdir_ref[...])


def kernel(x, ei, pos, feat, W0, b0, W1, b1, Wm1, bm1, Wm2, bm2,
           Wm3, bm3, Wdir, bdir):
    f32 = jnp.float32
    row = ei[0].reshape(NEB, 1, EBLK)
    col = ei[1].reshape(NEB, 1, EBLK)
    iq = jnp.concatenate([pos[:, 0].reshape(NQB, 1, HQB),
                          pos[:, 1].reshape(NQB, 1, HQB)], axis=2)
    jq = jnp.concatenate([pos[:, 1].reshape(NQB, 1, HQB),
                          pos[:, 0].reshape(NQB, 1, HQB)], axis=2)

    full = lambda shp: pl.BlockSpec(shp, lambda *_: tuple(0 for _ in shp))
    ebk = pl.BlockSpec((1, 1, EBLK), lambda e: (e, 0, 0))
    qbk = pl.BlockSpec((1, 1, QBLK), lambda q: (q, 0, 0))

    adjf, adjt, ti, tj, v1, u2, w = pl.pallas_call(
        _k1,
        grid=(NEB,),
        in_specs=[ebk, ebk, full((N, 128)), full((128, M)), full((1, M)),
                  full((M, M)), full((1, M)), full((2 * M, M)), full((1, M)),
                  full((2 * M, M)), full((1, M))],
        out_specs=[full((N, N)), full((N, N)), full((N, TD)), full((N, TD))]
        + [full((N, M))] * 3,
        out_shape=[jax.ShapeDtypeStruct((N, N), jnp.bfloat16)] * 2
        + [jax.ShapeDtypeStruct((N, TD), f32)] * 2
        + [jax.ShapeDtypeStruct((N, M), f32)] * 3,
    )(row, col, feat, W0, b0.reshape(1, M), W1, b1.reshape(1, M),
      Wm1, bm1.reshape(1, M), Wm2, bm2.reshape(1, M))

    sc = functools.partial(
        pl.kernel,
        out_type=[jax.ShapeDtypeStruct((2 * P, TD), f32)] * 2,
        mesh=plsc.VectorSubcoreMesh(core_axis_name="c", subcore_axis_name="s"),
        scratch_types=[pltpu.VMEM((_BPW,), jnp.int32),
                       pltpu.VMEM((_BPW,), jnp.int32),
                       pltpu.VMEM((_BPW, TD), f32),
                       pltpu.VMEM((_BPW, TD), f32),
                       pltpu.SemaphoreType.DMA,
                       pltpu.SemaphoreType.DMA],
    )(_sc_gather)
    gi, gj = sc(ti, tj, iq.reshape(-1), jq.reshape(-1))

    out = pl.pallas_call(
        _k2,
        grid=(NQB,),
        in_specs=[qbk,
                  pl.BlockSpec((QBLK, TD), lambda q: (q, 0)),
                  pl.BlockSpec((QBLK, TD), lambda q: (q, 0))]
        + [full((N, M))] * 3
        + [full((M + 1, M)), full((1, M)), full((2 * M, 1)), full((1, 1))],
        out_specs=pl.BlockSpec((HQB, 1), lambda q: (q, 0)),
        out_shape=jax.ShapeDtypeStruct((P, 1), f32),
    )(jq, gi, gj, v1, u2, w, Wm3, bm3.reshape(1, M),
      Wdir, bdir.reshape(1, 1))
    return out
